# Initial kernel scaffold; baseline (speedup 1.0000x reference)
import jax
import jax.numpy as jnp
from jax.experimental import pallas as pl


def kernel(x, table):
    raise NotImplementedError("write your pallas kernel here")


# speedup vs baseline: 4.7581x; 4.7581x over previous
import functools
import math

import jax
import jax.numpy as jnp
from jax import lax
from jax.experimental import pallas as pl
from jax.experimental.pallas import tpu as pltpu
from jax.experimental.pallas import tpu_sc as plsc

D_MODEL = 128
SCALE = math.sqrt(D_MODEL)


def _scale_body(t_ref, o_ref):
    o_ref[...] = t_ref[...] * SCALE


def _scale_table(table):
    v, d = table.shape
    blk = 2000
    assert v % blk == 0
    return pl.pallas_call(
        _scale_body,
        grid=(v // blk,),
        in_specs=[pl.BlockSpec((blk, d), lambda i: (i, 0))],
        out_specs=pl.BlockSpec((blk, d), lambda i: (i, 0)),
        out_shape=jax.ShapeDtypeStruct((v, d), jnp.float32),
    )(table)


@functools.cache
def _make_gather(v, d, b_total):
    info = plsc.get_sparse_core_info()
    nc, ns = info.num_cores, info.num_subcores
    nw = nc * ns
    b_per_w = b_total // nw
    chunk = 128
    n_chunks = b_per_w // chunk
    assert b_per_w % chunk == 0 and b_total % nw == 0

    mesh = plsc.VectorSubcoreMesh(core_axis_name="c", subcore_axis_name="s")

    @functools.partial(
        pl.kernel,
        mesh=mesh,
        out_type=jax.ShapeDtypeStruct((b_total, d), jnp.float32),
        scratch_types=[
            pltpu.VMEM((chunk,), jnp.int32),
            pltpu.VMEM((chunk, d), jnp.float32),
            pltpu.SemaphoreType.DMA,
        ],
    )
    def gather_kernel(table_hbm, idx_hbm, out_hbm, idx_v, rows_v, sem):
        wid = lax.axis_index("s") * nc + lax.axis_index("c")
        base = wid * b_per_w

        def body(j, carry):
            start = base + j * chunk
            pltpu.sync_copy(idx_hbm.at[pl.ds(start, chunk)], idx_v)
            pltpu.async_copy(table_hbm.at[idx_v], rows_v, sem).wait()
            pltpu.sync_copy(rows_v, out_hbm.at[pl.ds(start, chunk)])
            return carry

        lax.fori_loop(0, n_chunks, body, 0)

    return gather_kernel


def kernel(x, table):
    batch, seq = x.shape
    v, d = table.shape
    b_total = batch * seq
    scaled = _scale_table(table)
    flat_idx = x.reshape(b_total)
    out = _make_gather(v, d, b_total)(scaled, flat_idx)
    return out.reshape(batch, seq, d)


# speedup vs baseline: 7.9103x; 1.6625x over previous
import functools
import math

import jax
import jax.numpy as jnp
from jax import lax
from jax.experimental import pallas as pl
from jax.experimental.pallas import tpu as pltpu
from jax.experimental.pallas import tpu_sc as plsc

D_MODEL = 128
SCALE = math.sqrt(D_MODEL)


def _scale_body(t_ref, o_ref):
    o_ref[...] = t_ref[...] * SCALE


def _scale_table(table):
    v, d = table.shape
    blk = 2000
    assert v % blk == 0
    return pl.pallas_call(
        _scale_body,
        grid=(v // blk,),
        in_specs=[pl.BlockSpec((blk, d), lambda i: (i, 0))],
        out_specs=pl.BlockSpec((blk, d), lambda i: (i, 0)),
        out_shape=jax.ShapeDtypeStruct((v, d), jnp.float32),
    )(table)


@functools.cache
def _make_gather(v, d, b_total):
    info = plsc.get_sparse_core_info()
    nc, ns = info.num_cores, info.num_subcores
    nw = nc * ns
    b_per_w = b_total // nw
    chunk = 128
    n_chunks = b_per_w // chunk
    assert b_per_w % chunk == 0 and b_total % nw == 0

    nslots = 4
    n_outer = n_chunks // nslots
    assert n_chunks % nslots == 0

    mesh = plsc.VectorSubcoreMesh(core_axis_name="c", subcore_axis_name="s")

    @functools.partial(
        pl.kernel,
        mesh=mesh,
        out_type=jax.ShapeDtypeStruct((b_total, d), jnp.float32),
        scratch_types=[
            pltpu.VMEM((n_chunks, chunk), jnp.int32),
            pltpu.VMEM((nslots, chunk, d), jnp.float32),
        ]
        + [pltpu.SemaphoreType.DMA] * (2 * nslots),
    )
    def gather_kernel(table_hbm, idx_hbm, out_hbm, idx_v, rows_v, *sems):
        gsem = sems[:nslots]
        wsem = sems[nslots:]
        wid = lax.axis_index("s") * nc + lax.axis_index("c")
        base = wid * b_per_w

        pltpu.sync_copy(idx_hbm.at[wid], idx_v)

        def fire_gather(j, b):
            pltpu.async_copy(table_hbm.at[idx_v.at[j]], rows_v.at[b], gsem[b])

        def fire_write(j, b):
            pltpu.async_copy(rows_v.at[b],
                             out_hbm.at[pl.ds(base + j * chunk, chunk)],
                             wsem[b])

        def wait_gather(b):
            pltpu.make_async_copy(table_hbm.at[idx_v.at[0]], rows_v.at[b],
                                  gsem[b]).wait()

        def wait_write(b):
            pltpu.make_async_copy(rows_v.at[b],
                                  out_hbm.at[pl.ds(base, chunk)],
                                  wsem[b]).wait()

        for b in range(nslots):
            fire_gather(b, b)

        def body(i, carry):
            for b in range(nslots):
                wait_gather(b)
                fire_write(i * nslots + b, b)

            @pl.when(i < n_outer - 1)
            def _():
                for b in range(nslots):
                    wait_write(b)
                    fire_gather((i + 1) * nslots + b, b)

            return carry

        lax.fori_loop(0, n_outer, body, 0)
        for b in range(nslots):
            wait_write(b)

    def run(scaled_table, flat_idx):
        return gather_kernel(scaled_table,
                             flat_idx.reshape(nw, n_chunks, chunk))

    return run


def kernel(x, table):
    batch, seq = x.shape
    v, d = table.shape
    b_total = batch * seq
    scaled = _scale_table(table)
    flat_idx = x.reshape(b_total)
    out = _make_gather(v, d, b_total)(scaled, flat_idx)
    return out.reshape(batch, seq, d)


# speedup vs baseline: 9.0967x; 1.1500x over previous
import functools
import math

import jax
import jax.numpy as jnp
from jax import lax
from jax.experimental import pallas as pl
from jax.experimental.pallas import tpu as pltpu
from jax.experimental.pallas import tpu_sc as plsc

D_MODEL = 128
SCALE = math.sqrt(D_MODEL)


def _scale_body(t_ref, o_ref):
    o_ref[...] = t_ref[...] * SCALE


def _scale_table(table):
    v, d = table.shape
    blk = 2000
    assert v % blk == 0
    return pl.pallas_call(
        _scale_body,
        grid=(v // blk,),
        in_specs=[pl.BlockSpec((blk, d), lambda i: (i, 0))],
        out_specs=pl.BlockSpec((blk, d), lambda i: (i, 0)),
        out_shape=jax.ShapeDtypeStruct((v, d), jnp.float32),
    )(table)


@functools.cache
def _make_gather(v, d, b_total):
    info = plsc.get_sparse_core_info()
    nc, ns = info.num_cores, info.num_subcores
    nw = nc * ns
    b_per_w = b_total // nw
    chunk = 128
    n_chunks = b_per_w // chunk
    assert b_per_w % chunk == 0 and b_total % nw == 0

    nslots = 4
    n_outer = n_chunks // nslots
    assert n_chunks % nslots == 0

    mesh = plsc.VectorSubcoreMesh(core_axis_name="c", subcore_axis_name="s")

    @functools.partial(
        pl.kernel,
        mesh=mesh,
        out_type=jax.ShapeDtypeStruct((b_total, d), jnp.float32),
        scratch_types=[
            pltpu.VMEM((n_chunks, chunk), jnp.int32),
            pltpu.VMEM((nslots, chunk, d), jnp.float32),
        ]
        + [pltpu.SemaphoreType.DMA] * (2 * nslots),
    )
    def gather_kernel(table_hbm, idx_hbm, out_hbm, idx_v, rows_v, *sems):
        gsem = sems[:nslots]
        wsem = sems[nslots:]
        wid = lax.axis_index("s") * nc + lax.axis_index("c")
        base = wid * b_per_w

        pltpu.sync_copy(idx_hbm.at[wid], idx_v)

        def fire_gather(j, b):
            pltpu.async_copy(table_hbm.at[idx_v.at[j]], rows_v.at[b], gsem[b])

        def fire_write(j, b):
            pltpu.async_copy(rows_v.at[b],
                             out_hbm.at[pl.ds(base + j * chunk, chunk)],
                             wsem[b])

        def wait_gather(b):
            pltpu.make_async_copy(table_hbm.at[idx_v.at[0]], rows_v.at[b],
                                  gsem[b]).wait()

        def wait_write(b):
            pltpu.make_async_copy(rows_v.at[b],
                                  out_hbm.at[pl.ds(base, chunk)],
                                  wsem[b]).wait()

        def scale_slot(b):
            def srow(r, carry):
                for c in range(d // 16):
                    sl = pl.ds(c * 16, 16)
                    rows_v[b, r, sl] = rows_v[b, r, sl] * SCALE
                return carry

            lax.fori_loop(0, chunk, srow, 0)

        for b in range(nslots):
            fire_gather(b, b)

        def body(i, carry):
            for b in range(nslots):
                wait_gather(b)
                scale_slot(b)
                fire_write(i * nslots + b, b)

            @pl.when(i < n_outer - 1)
            def _():
                for b in range(nslots):
                    wait_write(b)
                    fire_gather((i + 1) * nslots + b, b)

            return carry

        lax.fori_loop(0, n_outer, body, 0)
        for b in range(nslots):
            wait_write(b)

    def run(scaled_table, flat_idx):
        return gather_kernel(scaled_table,
                             flat_idx.reshape(nw, n_chunks, chunk))

    return run


def kernel(x, table):
    batch, seq = x.shape
    v, d = table.shape
    b_total = batch * seq
    flat_idx = x.reshape(b_total)
    out = _make_gather(v, d, b_total)(table, flat_idx)
    return out.reshape(batch, seq, d)


# speedup vs baseline: 9.1223x; 1.0028x over previous
import functools
import math

import jax
import jax.numpy as jnp
from jax import lax
from jax.experimental import pallas as pl
from jax.experimental.pallas import tpu as pltpu
from jax.experimental.pallas import tpu_sc as plsc

D_MODEL = 128
SCALE = math.sqrt(D_MODEL)


def _scale_body(t_ref, o_ref):
    o_ref[...] = t_ref[...] * SCALE


def _scale_table(table):
    v, d = table.shape
    blk = 2000
    assert v % blk == 0
    return pl.pallas_call(
        _scale_body,
        grid=(v // blk,),
        in_specs=[pl.BlockSpec((blk, d), lambda i: (i, 0))],
        out_specs=pl.BlockSpec((blk, d), lambda i: (i, 0)),
        out_shape=jax.ShapeDtypeStruct((v, d), jnp.float32),
    )(table)


@functools.cache
def _make_gather(v, d, b_total):
    info = plsc.get_sparse_core_info()
    nc, ns = info.num_cores, info.num_subcores
    nw = nc * ns
    b_per_w = b_total // nw
    chunk = 128
    n_chunks = b_per_w // chunk
    assert b_per_w % chunk == 0 and b_total % nw == 0

    nslots = 5
    n_outer = n_chunks // nslots
    assert n_chunks % nslots == 0

    mesh = plsc.VectorSubcoreMesh(core_axis_name="c", subcore_axis_name="s")

    @functools.partial(
        pl.kernel,
        mesh=mesh,
        out_type=jax.ShapeDtypeStruct((b_total, d), jnp.float32),
        scratch_types=[
            pltpu.VMEM((n_chunks, chunk), jnp.int32),
            pltpu.VMEM((nslots, chunk, d), jnp.float32),
        ]
        + [pltpu.SemaphoreType.DMA] * (2 * nslots),
    )
    def gather_kernel(table_hbm, idx_hbm, out_hbm, idx_v, rows_v, *sems):
        gsem = sems[:nslots]
        wsem = sems[nslots:]
        wid = lax.axis_index("s") * nc + lax.axis_index("c")
        base = wid * b_per_w

        pltpu.sync_copy(idx_hbm.at[wid], idx_v)

        def fire_gather(j, b):
            pltpu.async_copy(table_hbm.at[idx_v.at[j]], rows_v.at[b], gsem[b])

        def fire_write(j, b):
            pltpu.async_copy(rows_v.at[b],
                             out_hbm.at[pl.ds(base + j * chunk, chunk)],
                             wsem[b])

        def wait_gather(b):
            pltpu.make_async_copy(table_hbm.at[idx_v.at[0]], rows_v.at[b],
                                  gsem[b]).wait()

        def wait_write(b):
            pltpu.make_async_copy(rows_v.at[b],
                                  out_hbm.at[pl.ds(base, chunk)],
                                  wsem[b]).wait()

        rows_per_it = 4

        def scale_slot(b):
            def srow(r, carry):
                for rr in range(rows_per_it):
                    for c in range(d // 16):
                        sl = pl.ds(c * 16, 16)
                        row = r * rows_per_it + rr
                        rows_v[b, row, sl] = rows_v[b, row, sl] * SCALE
                return carry

            lax.fori_loop(0, chunk // rows_per_it, srow, 0)

        for b in range(nslots):
            fire_gather(b, b)

        def body(i, carry):
            for b in range(nslots):
                wait_gather(b)
                scale_slot(b)
                fire_write(i * nslots + b, b)

            @pl.when(i < n_outer - 1)
            def _():
                for b in range(nslots):
                    wait_write(b)
                    fire_gather((i + 1) * nslots + b, b)

            return carry

        lax.fori_loop(0, n_outer, body, 0)
        for b in range(nslots):
            wait_write(b)

    def run(scaled_table, flat_idx):
        return gather_kernel(scaled_table,
                             flat_idx.reshape(nw, n_chunks, chunk))

    return run


def kernel(x, table):
    batch, seq = x.shape
    v, d = table.shape
    b_total = batch * seq
    flat_idx = x.reshape(b_total)
    out = _make_gather(v, d, b_total)(table, flat_idx)
    return out.reshape(batch, seq, d)


# speedup vs baseline: 9.1235x; 1.0001x over previous
import functools
import math

import jax
import jax.numpy as jnp
from jax import lax
from jax.experimental import pallas as pl
from jax.experimental.pallas import tpu as pltpu
from jax.experimental.pallas import tpu_sc as plsc

D_MODEL = 128
SCALE = math.sqrt(D_MODEL)


def _scale_body(t_ref, o_ref):
    o_ref[...] = t_ref[...] * SCALE


def _scale_table(table):
    v, d = table.shape
    blk = 2000
    assert v % blk == 0
    return pl.pallas_call(
        _scale_body,
        grid=(v // blk,),
        in_specs=[pl.BlockSpec((blk, d), lambda i: (i, 0))],
        out_specs=pl.BlockSpec((blk, d), lambda i: (i, 0)),
        out_shape=jax.ShapeDtypeStruct((v, d), jnp.float32),
    )(table)


@functools.cache
def _make_gather(v, d, b_total):
    info = plsc.get_sparse_core_info()
    nc, ns = info.num_cores, info.num_subcores
    nw = nc * ns
    b_per_w = b_total // nw
    chunk = 64
    n_chunks = b_per_w // chunk
    assert b_per_w % chunk == 0 and b_total % nw == 0

    nslots = 8
    n_outer = n_chunks // nslots
    assert n_chunks % nslots == 0

    mesh = plsc.VectorSubcoreMesh(core_axis_name="c", subcore_axis_name="s")

    @functools.partial(
        pl.kernel,
        mesh=mesh,
        out_type=jax.ShapeDtypeStruct((b_total, d), jnp.float32),
        scratch_types=[
            pltpu.VMEM((n_chunks, chunk), jnp.int32),
            pltpu.VMEM((nslots, chunk, d), jnp.float32),
        ]
        + [pltpu.SemaphoreType.DMA] * (2 * nslots),
    )
    def gather_kernel(table_hbm, idx_hbm, out_hbm, idx_v, rows_v, *sems):
        gsem = sems[:nslots]
        wsem = sems[nslots:]
        wid = lax.axis_index("s") * nc + lax.axis_index("c")
        base = wid * b_per_w

        pltpu.sync_copy(idx_hbm.at[wid], idx_v)

        def fire_gather(j, b):
            pltpu.async_copy(table_hbm.at[idx_v.at[j]], rows_v.at[b], gsem[b])

        def fire_write(j, b):
            pltpu.async_copy(rows_v.at[b],
                             out_hbm.at[pl.ds(base + j * chunk, chunk)],
                             wsem[b])

        def wait_gather(b):
            pltpu.make_async_copy(table_hbm.at[idx_v.at[0]], rows_v.at[b],
                                  gsem[b]).wait()

        def wait_write(b):
            pltpu.make_async_copy(rows_v.at[b],
                                  out_hbm.at[pl.ds(base, chunk)],
                                  wsem[b]).wait()

        rows_per_it = 4

        def scale_slot(b):
            def srow(r, carry):
                for rr in range(rows_per_it):
                    for c in range(d // 16):
                        sl = pl.ds(c * 16, 16)
                        row = r * rows_per_it + rr
                        rows_v[b, row, sl] = rows_v[b, row, sl] * SCALE
                return carry

            lax.fori_loop(0, chunk // rows_per_it, srow, 0)

        for b in range(nslots):
            fire_gather(b, b)

        def body(i, carry):
            for b in range(nslots):
                wait_gather(b)
                scale_slot(b)
                fire_write(i * nslots + b, b)

            @pl.when(i < n_outer - 1)
            def _():
                for b in range(nslots):
                    wait_write(b)
                    fire_gather((i + 1) * nslots + b, b)

            return carry

        lax.fori_loop(0, n_outer, body, 0)
        for b in range(nslots):
            wait_write(b)

    def run(scaled_table, flat_idx):
        return gather_kernel(scaled_table,
                             flat_idx.reshape(nw, n_chunks, chunk))

    return run


def kernel(x, table):
    batch, seq = x.shape
    v, d = table.shape
    b_total = batch * seq
    flat_idx = x.reshape(b_total)
    out = _make_gather(v, d, b_total)(table, flat_idx)
    return out.reshape(batch, seq, d)
